# baseline (speedup 1.0000x reference)
import functools

import jax
import jax.numpy as jnp
from jax import lax
from jax.experimental import pallas as pl
from jax.experimental.pallas import tpu as pltpu
from jax.experimental.pallas import tpu_sc as plsc

EMB = 128
PHASES = 4


@functools.cache
def _build_gather(B, T, phase):
    info = plsc.get_sparse_core_info()
    NC, NS = info.num_cores, info.num_subcores
    NW = NC * NS
    PB = B // PHASES
    b_per_w = PB // NW
    NB = 2
    G = b_per_w // NB
    assert PB % NW == 0 and b_per_w % NB == 0 and G % 2 == 0 and G >= 6
    phase0 = phase * PB

    mesh = plsc.VectorSubcoreMesh(core_axis_name="c", subcore_axis_name="s")

    @functools.partial(
        pl.kernel,
        mesh=mesh,
        out_type=jax.ShapeDtypeStruct((PB, T, EMB), jnp.float32),
        scratch_types=[
            pltpu.VMEM((2, NB, T), jnp.int32),
            pltpu.VMEM((2, NB, T, EMB), jnp.float32),
            pltpu.SemaphoreType.DMA,
            pltpu.SemaphoreType.DMA,
            pltpu.SemaphoreType.DMA,
            pltpu.SemaphoreType.DMA,
            pltpu.SemaphoreType.DMA,
            pltpu.SemaphoreType.DMA,
        ],
    )
    def gather_kernel(x_hbm, table_hbm, out_hbm, idx_v, rows_v,
                      gsem0, gsem1, osem0, osem1, isem0, isem1):
        gsem = (gsem0, gsem1)
        osem = (osem0, osem1)
        isem = (isem0, isem1)
        wid = lax.axis_index("s") * NC + lax.axis_index("c")
        base = wid * b_per_w
        xbase = phase0 + base

        def start_idx(c, u):
            pltpu.async_copy(
                x_hbm.at[pl.ds(xbase + c * NB, NB)], idx_v.at[u], isem[u]
            )

        def wait_idx(u):
            pltpu.make_async_copy(
                x_hbm.at[pl.ds(xbase, NB)], idx_v.at[u], isem[u]
            ).wait()

        def start_gather(c, u):
            for j in range(NB):
                pltpu.async_copy(
                    table_hbm.at[idx_v.at[u, j]],
                    rows_v.at[u, j],
                    gsem[u],
                )

        def wait_gather(u):
            pltpu.make_async_copy(
                out_hbm.at[pl.ds(0, NB)], rows_v.at[u], gsem[u]
            ).wait()

        def start_store(c, u):
            pltpu.async_copy(
                rows_v.at[u], out_hbm.at[pl.ds(base + c * NB, NB)], osem[u]
            )

        def wait_store(u):
            pltpu.make_async_copy(
                rows_v.at[u], out_hbm.at[pl.ds(base, NB)], osem[u]
            ).wait()

        start_idx(0, 0)
        start_idx(1, 1)
        wait_idx(0)
        start_gather(0, 0)
        wait_gather(0)
        start_idx(2, 0)
        start_store(0, 0)
        wait_idx(1)
        start_gather(1, 1)

        def body(t, carry):
            for i in range(2):
                c = 1 + 2 * t + i
                u = 1 - i
                wait_gather(u)
                start_idx(c + 2, u)
                start_store(c, u)
                wait_store(1 - u)
                wait_idx(1 - u)
                start_gather(c + 1, 1 - u)
            return carry

        lax.fori_loop(0, (G - 2) // 2 - 1, body, 0)

        for i in range(2):
            c = G - 3 + i
            u = 1 - i
            wait_gather(u)
            if c + 2 < G:
                start_idx(c + 2, u)
            start_store(c, u)
            wait_store(1 - u)
            wait_idx(1 - u)
            start_gather(c + 1, 1 - u)

        wait_gather(1)
        start_store(G - 1, 1)
        wait_store(0)
        wait_store(1)

    return gather_kernel


def kernel(x, encoding):
    B, T = x.shape
    xi = x.astype(jnp.int32)
    parts = [_build_gather(B, T, p)(xi, encoding) for p in range(PHASES)]
    return jnp.concatenate(parts, axis=0)


# speedup vs baseline: 1.0525x; 1.0525x over previous
import functools

import jax
import jax.numpy as jnp
from jax import lax
from jax.experimental import pallas as pl
from jax.experimental.pallas import tpu as pltpu
from jax.experimental.pallas import tpu_sc as plsc

EMB = 128
PHASES = 4


@functools.cache
def _build_gather(B, T, phase):
    info = plsc.get_sparse_core_info()
    NC, NS = info.num_cores, info.num_subcores
    NW = NC * NS
    PB = B // PHASES
    b_per_w = PB // NW
    NB = 2
    G = b_per_w // NB
    assert PB % NW == 0 and b_per_w % NB == 0 and G % 2 == 0 and G >= 6
    phase0 = phase * PB

    mesh = plsc.VectorSubcoreMesh(core_axis_name="c", subcore_axis_name="s")

    @functools.partial(
        pl.kernel,
        mesh=mesh,
        out_type=jax.ShapeDtypeStruct((PB, T, EMB), jnp.float32),
        scratch_types=[
            pltpu.VMEM((2, NB, T), jnp.int32),
            pltpu.VMEM((2, NB, T, EMB), jnp.float32),
            pltpu.SemaphoreType.DMA,
            pltpu.SemaphoreType.DMA,
            pltpu.SemaphoreType.DMA,
            pltpu.SemaphoreType.DMA,
            pltpu.SemaphoreType.DMA,
            pltpu.SemaphoreType.DMA,
        ],
    )
    def gather_kernel(x_hbm, table_hbm, out_hbm, idx_v, rows_v,
                      gsem0, gsem1, osem0, osem1, isem0, isem1):
        gsem = (gsem0, gsem1)
        osem = (osem0, osem1)
        isem = (isem0, isem1)
        wid = lax.axis_index("s") * NC + lax.axis_index("c")
        base = wid * b_per_w
        xbase = phase0 + base

        def start_idx(c, u):
            pltpu.async_copy(
                x_hbm.at[pl.ds(xbase + c * NB, NB)], idx_v.at[u], isem[u]
            )

        def wait_idx(u):
            pltpu.make_async_copy(
                x_hbm.at[pl.ds(xbase, NB)], idx_v.at[u], isem[u]
            ).wait()

        def start_gather(c, u):
            for j in range(NB):
                pltpu.async_copy(
                    table_hbm.at[idx_v.at[u, j]],
                    rows_v.at[u, j],
                    gsem[u],
                )

        def wait_gather(u):
            pltpu.make_async_copy(
                out_hbm.at[pl.ds(0, NB)], rows_v.at[u], gsem[u]
            ).wait()

        def start_store(c, u):
            pltpu.async_copy(
                rows_v.at[u], out_hbm.at[pl.ds(base + c * NB, NB)], osem[u]
            )

        def wait_store(u):
            pltpu.make_async_copy(
                rows_v.at[u], out_hbm.at[pl.ds(base, NB)], osem[u]
            ).wait()

        start_idx(0, 0)
        start_idx(1, 1)
        wait_idx(0)
        start_gather(0, 0)
        wait_gather(0)
        start_idx(2, 0)
        start_store(0, 0)
        wait_idx(1)
        start_gather(1, 1)

        def body(t, carry):
            for i in range(2):
                c = 1 + 2 * t + i
                u = 1 - i
                wait_gather(u)
                start_idx(c + 2, u)
                start_store(c, u)
                wait_store(1 - u)
                wait_idx(1 - u)
                start_gather(c + 1, 1 - u)
            return carry

        lax.fori_loop(0, (G - 2) // 2 - 1, body, 0)

        for i in range(2):
            c = G - 3 + i
            u = 1 - i
            wait_gather(u)
            if c + 2 < G:
                start_idx(c + 2, u)
            start_store(c, u)
            wait_store(1 - u)
            wait_idx(1 - u)
            start_gather(c + 1, 1 - u)

        wait_gather(1)
        start_store(G - 1, 1)
        wait_store(0)
        wait_store(1)

    return gather_kernel


def kernel(x, encoding):
    B, T = x.shape
    xi = x.astype(jnp.int32)
    parts = [_build_gather(B, T, p)(xi, encoding) for p in range(PHASES)]
    out = jnp.zeros((B, T, EMB), jnp.float32)
    for p in range(PHASES):
        out = lax.dynamic_update_slice(out, parts[p], (p * (B // PHASES), 0, 0))
    return out


# speedup vs baseline: 1.9545x; 1.8570x over previous
import functools

import jax
import jax.numpy as jnp
from jax import lax
from jax.experimental import pallas as pl
from jax.experimental.pallas import tpu as pltpu
from jax.experimental.pallas import tpu_sc as plsc

EMB = 128


@functools.cache
def _build_gather(B, T):
    info = plsc.get_sparse_core_info()
    NC, NS = info.num_cores, info.num_subcores
    NW = NC * NS
    b_per_w = B // NW
    NB = 8
    G = b_per_w // NB
    assert B % NW == 0 and b_per_w % NB == 0 and G % 2 == 0 and G >= 6

    mesh = plsc.VectorSubcoreMesh(core_axis_name="c", subcore_axis_name="s")

    @functools.partial(
        pl.kernel,
        mesh=mesh,
        out_type=jax.ShapeDtypeStruct((B, T, EMB), jnp.float32),
        scratch_types=[
            pltpu.VMEM((2, NB, T), jnp.int32),
            pltpu.VMEM((2, NB, T, EMB), jnp.float32),
            pltpu.SemaphoreType.DMA,
            pltpu.SemaphoreType.DMA,
            pltpu.SemaphoreType.DMA,
            pltpu.SemaphoreType.DMA,
            pltpu.SemaphoreType.DMA,
            pltpu.SemaphoreType.DMA,
        ],
    )
    def gather_kernel(x_hbm, table_hbm, out_hbm, idx_v, rows_v,
                      gsem0, gsem1, osem0, osem1, isem0, isem1):
        gsem = (gsem0, gsem1)
        osem = (osem0, osem1)
        isem = (isem0, isem1)
        wid = lax.axis_index("s") * NC + lax.axis_index("c")
        base = wid * b_per_w

        def start_idx(c, u):
            pltpu.async_copy(
                x_hbm.at[pl.ds(base + c * NB, NB)], idx_v.at[u], isem[u]
            )

        def wait_idx(u):
            pltpu.make_async_copy(
                x_hbm.at[pl.ds(base, NB)], idx_v.at[u], isem[u]
            ).wait()

        def start_gather(c, u):
            for j in range(NB):
                pltpu.async_copy(
                    table_hbm.at[idx_v.at[u, j]],
                    rows_v.at[u, j],
                    gsem[u],
                )

        def wait_gather(u):
            pltpu.make_async_copy(
                out_hbm.at[pl.ds(0, NB)], rows_v.at[u], gsem[u]
            ).wait()

        def start_store(c, u):
            pltpu.async_copy(
                rows_v.at[u], out_hbm.at[pl.ds(base + c * NB, NB)], osem[u]
            )

        def wait_store(u):
            pltpu.make_async_copy(
                rows_v.at[u], out_hbm.at[pl.ds(base, NB)], osem[u]
            ).wait()

        start_idx(0, 0)
        start_idx(1, 1)
        wait_idx(0)
        start_gather(0, 0)
        wait_gather(0)
        start_idx(2, 0)
        start_store(0, 0)
        wait_idx(1)
        start_gather(1, 1)

        def body(t, carry):
            for i in range(2):
                c = 1 + 2 * t + i
                u = 1 - i
                wait_gather(u)
                start_idx(c + 2, u)
                start_store(c, u)
                wait_store(1 - u)
                wait_idx(1 - u)
                start_gather(c + 1, 1 - u)
            return carry

        lax.fori_loop(0, (G - 2) // 2 - 1, body, 0)

        for i in range(2):
            c = G - 3 + i
            u = 1 - i
            wait_gather(u)
            if c + 2 < G:
                start_idx(c + 2, u)
            start_store(c, u)
            wait_store(1 - u)
            wait_idx(1 - u)
            start_gather(c + 1, 1 - u)

        wait_gather(1)
        start_store(G - 1, 1)
        wait_store(0)
        wait_store(1)

    return gather_kernel


def kernel(x, encoding):
    B, T = x.shape
    return _build_gather(B, T)(x.astype(jnp.int32), encoding)
